# Initial kernel scaffold; baseline (speedup 1.0000x reference)
import jax
import jax.numpy as jnp
from jax.experimental import pallas as pl


def kernel(x, edge_index, edge_attr, W_in, b_in, Wm1, bm1, Wm2, bm2, Wm3, bm3, W_ih, W_hh, b_ih, b_hh, W_out, b_out):
    raise NotImplementedError("write your pallas kernel here")


# speedup vs baseline: 2.5505x; 2.5505x over previous
import functools

import jax
import jax.numpy as jnp
from jax import lax
from jax.experimental import pallas as pl
from jax.experimental.pallas import tpu as pltpu
from jax.experimental.pallas import tpu_sc as plsc

N_ = 10000
E_ = 320000
H_ = 128
DE_ = 16
STEPS_ = 3

NW = 32
EPW = E_ // NW
CHUNK = 80
CPW = EPW // CHUNK
N_PAD = 10240
TILE_ROWS = N_PAD // 16
ZROWS = 128

NB = 1000
EB = 1280

f32 = jnp.float32


def _init_body(x, w_in, b_in, w1i, w1j, h_o, a_o, b_o):
    h = jnp.dot(x[...], w_in[...], preferred_element_type=f32) + b_in[...]
    h_o[...] = h
    a_o[...] = jnp.dot(h, w1i[...], preferred_element_type=f32)
    b_o[...] = jnp.dot(h, w1j[...], preferred_element_type=f32)


def _mlp_body(ga, gb, ea, w1e, b1, w2, b2, w3, b3, v_o):
    t = ga[...] + gb[...] + jnp.dot(ea[...], w1e[...], preferred_element_type=f32) + b1[...]
    t = jnp.maximum(t, 0.0)
    u = jnp.maximum(jnp.dot(t, w2[...], preferred_element_type=f32) + b2[...], 0.0)
    v_o[...] = jnp.dot(u, w3[...], preferred_element_type=f32) + b3[...]


def _gru_body(p0, p1, h, w_ih, w_hh, b_ih, b_hh, w1i, w1j, h_o, a_o, b_o):
    agg = p0[...] + p1[...]
    hh = h[...]
    gi = jnp.dot(agg, w_ih[...], preferred_element_type=f32) + b_ih[...]
    gh = jnp.dot(hh, w_hh[...], preferred_element_type=f32) + b_hh[...]
    r = jax.nn.sigmoid(gi[:, :H_] + gh[:, :H_])
    z = jax.nn.sigmoid(gi[:, H_:2 * H_] + gh[:, H_:2 * H_])
    n = jnp.tanh(gi[:, 2 * H_:] + r * gh[:, 2 * H_:])
    hn = (1.0 - z) * n + z * hh
    h_o[...] = hn
    a_o[...] = jnp.dot(hn, w1i[...], preferred_element_type=f32)
    b_o[...] = jnp.dot(hn, w1j[...], preferred_element_type=f32)


def _head_body(h, w_out, b_out, o):
    o[...] = jnp.dot(h[...], w_out[...], preferred_element_type=f32) + b_out[...]


def _sc_gather_body(a_t, b_t, dst_h, src_h, ga_h, gb_h, idx_d, idx_s, abuf, bbuf):
    cid = lax.axis_index("c")
    sid = lax.axis_index("s")
    wid = sid * 2 + cid
    base = wid * EPW
    pltpu.sync_copy(dst_h.at[pl.ds(base, EPW)], idx_d)
    pltpu.sync_copy(src_h.at[pl.ds(base, EPW)], idx_s)

    def body(j, carry):
        off = j * CHUNK
        pltpu.sync_copy(a_t.at[idx_d.at[pl.ds(off, CHUNK)]], abuf)
        pltpu.sync_copy(b_t.at[idx_s.at[pl.ds(off, CHUNK)]], bbuf)
        pltpu.sync_copy(abuf, ga_h.at[pl.ds(base + off, CHUNK)])
        pltpu.sync_copy(bbuf, gb_h.at[pl.ds(base + off, CHUNK)])
        return carry

    lax.fori_loop(0, CPW, body, 0)


def _sc_scatter_body(v_h, dst_h, p0_h, p1_h, idx_c, vbuf, zbuf, shared):
    cid = lax.axis_index("c")
    sid = lax.axis_index("s")
    wid = sid * 2 + cid
    base = wid * EPW

    def zfill(i, carry):
        for k in range(8):
            zbuf[i, pl.ds(k * 16, 16)] = jnp.zeros((16,), f32)
        return carry

    lax.fori_loop(0, ZROWS, zfill, 0)
    for t in range(TILE_ROWS // ZROWS):
        pltpu.sync_copy(zbuf, shared.at[pl.ds(sid * TILE_ROWS + t * ZROWS, ZROWS)])
    plsc.subcore_barrier()

    def body(j, carry):
        off = j * CHUNK
        pltpu.sync_copy(v_h.at[pl.ds(base + off, CHUNK)], vbuf)
        pltpu.sync_copy(dst_h.at[pl.ds(base + off, CHUNK)], idx_c)
        pltpu.sync_copy(vbuf, shared.at[idx_c], add=True)
        return carry

    lax.fori_loop(0, CPW, body, 0)
    plsc.subcore_barrier()

    @pl.when(cid == 0)
    def _():
        for t in range(TILE_ROWS // ZROWS):
            row = sid * TILE_ROWS + t * ZROWS
            pltpu.sync_copy(shared.at[pl.ds(row, ZROWS)], zbuf)
            pltpu.sync_copy(zbuf, p0_h.at[pl.ds(row, ZROWS)])

    @pl.when(cid == 1)
    def _():
        for t in range(TILE_ROWS // ZROWS):
            row = sid * TILE_ROWS + t * ZROWS
            pltpu.sync_copy(shared.at[pl.ds(row, ZROWS)], zbuf)
            pltpu.sync_copy(zbuf, p1_h.at[pl.ds(row, ZROWS)])


def kernel(x, edge_index, edge_attr, W_in, b_in, Wm1, bm1, Wm2, bm2, Wm3, bm3,
           W_ih, W_hh, b_ih, b_hh, W_out, b_out):
    src1 = edge_index[0]
    dst1 = edge_index[1]
    w1i = Wm1[:H_]
    w1j = Wm1[H_:2 * H_]
    w1e = Wm1[2 * H_:]
    b1 = bm1.reshape(1, H_)
    b2 = bm2.reshape(1, H_ // 2)
    b3 = bm3.reshape(1, H_)
    bih = b_ih.reshape(1, 3 * H_)
    bhh = b_hh.reshape(1, 3 * H_)
    bo = b_out.reshape(1, 1)

    grid_n = N_ // NB
    grid_e = E_ // EB

    def full(s):
        return pl.BlockSpec(s, lambda i: (0, 0))

    rowblk = pl.BlockSpec((NB, H_), lambda i: (i, 0))

    init_call = pl.pallas_call(
        _init_body,
        grid=(grid_n,),
        in_specs=[rowblk, full((H_, H_)), full((1, H_)), full((H_, H_)), full((H_, H_))],
        out_specs=[rowblk, rowblk, rowblk],
        out_shape=[jax.ShapeDtypeStruct((N_, H_), f32)] * 3,
    )
    h, A, B = init_call(x, W_in, b_in.reshape(1, H_), w1i, w1j)

    edgeblk = pl.BlockSpec((EB, H_), lambda i: (i, 0))
    mlp_call = pl.pallas_call(
        _mlp_body,
        grid=(grid_e,),
        in_specs=[edgeblk, edgeblk, pl.BlockSpec((EB, DE_), lambda i: (i, 0)),
                  full((DE_, H_)), full((1, H_)), full((H_, H_ // 2)),
                  full((1, H_ // 2)), full((H_ // 2, H_)), full((1, H_))],
        out_specs=[edgeblk],
        out_shape=[jax.ShapeDtypeStruct((E_, H_), f32)],
    )

    gru_call = pl.pallas_call(
        _gru_body,
        grid=(grid_n,),
        in_specs=[rowblk, rowblk, rowblk, full((H_, 3 * H_)), full((H_, 3 * H_)),
                  full((1, 3 * H_)), full((1, 3 * H_)), full((H_, H_)), full((H_, H_))],
        out_specs=[rowblk, rowblk, rowblk],
        out_shape=[jax.ShapeDtypeStruct((N_, H_), f32)] * 3,
    )

    mesh = plsc.VectorSubcoreMesh(core_axis_name="c", subcore_axis_name="s")
    gather_call = pl.kernel(
        _sc_gather_body,
        out_type=(jax.ShapeDtypeStruct((E_, H_), f32),
                  jax.ShapeDtypeStruct((E_, H_), f32)),
        mesh=mesh,
        scratch_types=[
            pltpu.VMEM((EPW,), jnp.int32),
            pltpu.VMEM((EPW,), jnp.int32),
            pltpu.VMEM((CHUNK, H_), f32),
            pltpu.VMEM((CHUNK, H_), f32),
        ],
    )
    scatter_call = pl.kernel(
        _sc_scatter_body,
        out_type=(jax.ShapeDtypeStruct((N_PAD, H_), f32),
                  jax.ShapeDtypeStruct((N_PAD, H_), f32)),
        mesh=mesh,
        scratch_types=[
            pltpu.VMEM((CHUNK,), jnp.int32),
            pltpu.VMEM((CHUNK, H_), f32),
            pltpu.VMEM((ZROWS, H_), f32),
            pltpu.VMEM_SHARED((N_PAD, H_), f32),
        ],
    )

    for _ in range(STEPS_):
        GA, GB = gather_call(A, B, dst1, src1)
        (v,) = mlp_call(GA, GB, edge_attr, w1e, b1, Wm2, b2, Wm3, b3)
        p0, p1 = scatter_call(v, dst1)
        h, A, B = gru_call(p0, p1, h, W_ih, W_hh, bih, bhh, w1i, w1j)

    head_call = pl.pallas_call(
        _head_body,
        grid=(grid_n,),
        in_specs=[rowblk, full((H_, 1)), full((1, 1))],
        out_specs=[pl.BlockSpec((NB, 1), lambda i: (i, 0))],
        out_shape=[jax.ShapeDtypeStruct((N_, 1), f32)],
    )
    (out,) = head_call(h, W_out, bo)
    return out


# speedup vs baseline: 4.0523x; 1.5888x over previous
import functools

import jax
import jax.numpy as jnp
from jax import lax
from jax.experimental import pallas as pl
from jax.experimental.pallas import tpu as pltpu
from jax.experimental.pallas import tpu_sc as plsc

N_ = 10000
E_ = 320000
H_ = 128
DE_ = 16
STEPS_ = 3

NW = 32
EPW = E_ // NW
CHUNK = 80
CPW = EPW // CHUNK
N_PAD = 10240
TILE_ROWS = N_PAD // 16
ZROWS = 128

NB = 1000
EB = 1280

f32 = jnp.float32


def _init_body(x, w_in, b_in, w1i, w1j, h_o, a_o, b_o):
    h = jnp.dot(x[...], w_in[...], preferred_element_type=f32) + b_in[...]
    h_o[...] = h
    a_o[...] = jnp.dot(h, w1i[...], preferred_element_type=f32)
    b_o[...] = jnp.dot(h, w1j[...], preferred_element_type=f32)


def _mlp_body(g, ea, w1e, b1, w2, b2, w3, b3, v_o):
    t = g[...] + jnp.dot(ea[...], w1e[...], preferred_element_type=f32) + b1[...]
    t = jnp.maximum(t, 0.0)
    u = jnp.maximum(jnp.dot(t, w2[...], preferred_element_type=f32) + b2[...], 0.0)
    v_o[...] = jnp.dot(u, w3[...], preferred_element_type=f32) + b3[...]


def _gru_body(p0, p1, h, w_ih, w_hh, b_ih, b_hh, w1i, w1j, h_o, a_o, b_o):
    agg = p0[...] + p1[...]
    hh = h[...]
    gi = jnp.dot(agg, w_ih[...], preferred_element_type=f32) + b_ih[...]
    gh = jnp.dot(hh, w_hh[...], preferred_element_type=f32) + b_hh[...]
    r = jax.nn.sigmoid(gi[:, :H_] + gh[:, :H_])
    z = jax.nn.sigmoid(gi[:, H_:2 * H_] + gh[:, H_:2 * H_])
    n = jnp.tanh(gi[:, 2 * H_:] + r * gh[:, 2 * H_:])
    hn = (1.0 - z) * n + z * hh
    h_o[...] = hn
    a_o[...] = jnp.dot(hn, w1i[...], preferred_element_type=f32)
    b_o[...] = jnp.dot(hn, w1j[...], preferred_element_type=f32)


def _head_body(h, w_out, b_out, o):
    o[...] = jnp.dot(h[...], w_out[...], preferred_element_type=f32) + b_out[...]


def _sc_gather_body(a_t, b_t, dst_h, src_h, g_h,
                    idx_d, idx_s, abuf0, abuf1, bbuf0, bbuf1,
                    sg0, sg1, so0, so1):
    cid = lax.axis_index("c")
    sid = lax.axis_index("s")
    wid = sid * 2 + cid
    base = wid * EPW
    pltpu.sync_copy(dst_h.at[pl.ds(base, EPW)], idx_d)
    pltpu.sync_copy(src_h.at[pl.ds(base, EPW)], idx_s)

    abufs = (abuf0, abuf1)
    bbufs = (bbuf0, bbuf1)
    sgs = (sg0, sg1)
    sos = (so0, so1)

    def issue(j, p):
        off = j * CHUNK
        pltpu.async_copy(a_t.at[idx_d.at[pl.ds(off, CHUNK)]], abufs[p], sgs[p])
        pltpu.async_copy(b_t.at[idx_s.at[pl.ds(off, CHUNK)]], bbufs[p], sgs[p])

    def wait_gathers(j, p):
        off = j * CHUNK
        pltpu.make_async_copy(a_t.at[idx_d.at[pl.ds(off, CHUNK)]], abufs[p], sgs[p]).wait()
        pltpu.make_async_copy(b_t.at[idx_s.at[pl.ds(off, CHUNK)]], bbufs[p], sgs[p]).wait()

    def drain_out(p):
        pltpu.make_async_copy(abufs[p], g_h.at[pl.ds(base, CHUNK)], sos[p]).wait()

    def add_rows(p):
        ab, bb = abufs[p], bbufs[p]

        def row(i, carry):
            for k in range(8):
                sl = pl.ds(k * 16, 16)
                ab[i, sl] = ab[i, sl] + bb[i, sl]
            return carry

        lax.fori_loop(0, CHUNK, row, 0)

    issue(0, 0)

    def outer(g, carry):
        for b in range(2):
            j = 2 * g + b
            nb = 1 - b

            @pl.when(jnp.logical_or(g >= 1, b == 1))
            def _():
                drain_out(nb)

            issue(j + 1, nb)
            wait_gathers(j, b)
            add_rows(b)
            pltpu.async_copy(abufs[b], g_h.at[pl.ds(base + j * CHUNK, CHUNK)], sos[b])
        return carry

    lax.fori_loop(0, (CPW - 1) // 2, outer, 0)

    j_last = CPW - 1
    wait_gathers(j_last, 0)
    add_rows(0)
    pltpu.sync_copy(abufs[0], g_h.at[pl.ds(base + j_last * CHUNK, CHUNK)])
    drain_out(1)


def _sc_scatter_body(v_h, dst_h, p0_h, p1_h,
                     idx0, idx1, vbuf0, vbuf1, zbuf, shared,
                     sv0, sv1, sa0, sa1):
    cid = lax.axis_index("c")
    sid = lax.axis_index("s")
    wid = sid * 2 + cid
    base = wid * EPW

    idxs = (idx0, idx1)
    vbufs = (vbuf0, vbuf1)
    svs = (sv0, sv1)
    sas = (sa0, sa1)

    def issue_reads(j, p):
        off = base + j * CHUNK
        pltpu.async_copy(v_h.at[pl.ds(off, CHUNK)], vbufs[p], svs[p])
        pltpu.async_copy(dst_h.at[pl.ds(off, CHUNK)], idxs[p], svs[p])

    def wait_reads(j, p):
        off = base + j * CHUNK
        pltpu.make_async_copy(v_h.at[pl.ds(off, CHUNK)], vbufs[p], svs[p]).wait()
        pltpu.make_async_copy(dst_h.at[pl.ds(off, CHUNK)], idxs[p], svs[p]).wait()

    def drain_add(p):
        pltpu.make_async_copy(vbufs[p], shared.at[idxs[p]], sas[p]).wait()

    def zfill(i, carry):
        for k in range(8):
            zbuf[i, pl.ds(k * 16, 16)] = jnp.zeros((16,), f32)
        return carry

    lax.fori_loop(0, ZROWS, zfill, 0)
    issue_reads(0, 0)
    for t in range(TILE_ROWS // ZROWS):
        pltpu.sync_copy(zbuf, shared.at[pl.ds(sid * TILE_ROWS + t * ZROWS, ZROWS)])
    plsc.subcore_barrier()

    def outer(g, carry):
        for b in range(2):
            j = 2 * g + b
            nb = 1 - b

            @pl.when(jnp.logical_or(g >= 1, b == 1))
            def _():
                drain_add(nb)

            issue_reads(j + 1, nb)
            wait_reads(j, b)
            pltpu.async_copy(vbufs[b], shared.at[idxs[b]], sas[b], add=True)
        return carry

    lax.fori_loop(0, (CPW - 1) // 2, outer, 0)

    j_last = CPW - 1
    drain_add(1)
    wait_reads(j_last, 0)
    pltpu.sync_copy(vbufs[0], shared.at[idxs[0]], add=True)
    plsc.subcore_barrier()

    @pl.when(cid == 0)
    def _():
        for t in range(TILE_ROWS // ZROWS):
            row = sid * TILE_ROWS + t * ZROWS
            pltpu.sync_copy(shared.at[pl.ds(row, ZROWS)], zbuf)
            pltpu.sync_copy(zbuf, p0_h.at[pl.ds(row, ZROWS)])

    @pl.when(cid == 1)
    def _():
        for t in range(TILE_ROWS // ZROWS):
            row = sid * TILE_ROWS + t * ZROWS
            pltpu.sync_copy(shared.at[pl.ds(row, ZROWS)], zbuf)
            pltpu.sync_copy(zbuf, p1_h.at[pl.ds(row, ZROWS)])


def kernel(x, edge_index, edge_attr, W_in, b_in, Wm1, bm1, Wm2, bm2, Wm3, bm3,
           W_ih, W_hh, b_ih, b_hh, W_out, b_out):
    src1 = edge_index[0]
    dst1 = edge_index[1]
    w1i = Wm1[:H_]
    w1j = Wm1[H_:2 * H_]
    w1e = Wm1[2 * H_:]
    b1 = bm1.reshape(1, H_)
    b2 = bm2.reshape(1, H_ // 2)
    b3 = bm3.reshape(1, H_)
    bih = b_ih.reshape(1, 3 * H_)
    bhh = b_hh.reshape(1, 3 * H_)
    bo = b_out.reshape(1, 1)

    grid_n = N_ // NB
    grid_e = E_ // EB

    def full(s):
        return pl.BlockSpec(s, lambda i: (0, 0))

    rowblk = pl.BlockSpec((NB, H_), lambda i: (i, 0))

    init_call = pl.pallas_call(
        _init_body,
        grid=(grid_n,),
        in_specs=[rowblk, full((H_, H_)), full((1, H_)), full((H_, H_)), full((H_, H_))],
        out_specs=[rowblk, rowblk, rowblk],
        out_shape=[jax.ShapeDtypeStruct((N_, H_), f32)] * 3,
    )
    h, A, B = init_call(x, W_in, b_in.reshape(1, H_), w1i, w1j)

    edgeblk = pl.BlockSpec((EB, H_), lambda i: (i, 0))
    mlp_call = pl.pallas_call(
        _mlp_body,
        grid=(grid_e,),
        in_specs=[edgeblk, pl.BlockSpec((EB, DE_), lambda i: (i, 0)),
                  full((DE_, H_)), full((1, H_)), full((H_, H_ // 2)),
                  full((1, H_ // 2)), full((H_ // 2, H_)), full((1, H_))],
        out_specs=[edgeblk],
        out_shape=[jax.ShapeDtypeStruct((E_, H_), f32)],
    )

    gru_call = pl.pallas_call(
        _gru_body,
        grid=(grid_n,),
        in_specs=[rowblk, rowblk, rowblk, full((H_, 3 * H_)), full((H_, 3 * H_)),
                  full((1, 3 * H_)), full((1, 3 * H_)), full((H_, H_)), full((H_, H_))],
        out_specs=[rowblk, rowblk, rowblk],
        out_shape=[jax.ShapeDtypeStruct((N_, H_), f32)] * 3,
    )

    mesh = plsc.VectorSubcoreMesh(core_axis_name="c", subcore_axis_name="s")
    gather_call = pl.kernel(
        _sc_gather_body,
        out_type=jax.ShapeDtypeStruct((E_, H_), f32),
        mesh=mesh,
        scratch_types=[
            pltpu.VMEM((EPW,), jnp.int32),
            pltpu.VMEM((EPW,), jnp.int32),
            pltpu.VMEM((CHUNK, H_), f32),
            pltpu.VMEM((CHUNK, H_), f32),
            pltpu.VMEM((CHUNK, H_), f32),
            pltpu.VMEM((CHUNK, H_), f32),
            pltpu.SemaphoreType.DMA,
            pltpu.SemaphoreType.DMA,
            pltpu.SemaphoreType.DMA,
            pltpu.SemaphoreType.DMA,
        ],
    )
    scatter_call = pl.kernel(
        _sc_scatter_body,
        out_type=(jax.ShapeDtypeStruct((N_PAD, H_), f32),
                  jax.ShapeDtypeStruct((N_PAD, H_), f32)),
        mesh=mesh,
        scratch_types=[
            pltpu.VMEM((CHUNK,), jnp.int32),
            pltpu.VMEM((CHUNK,), jnp.int32),
            pltpu.VMEM((CHUNK, H_), f32),
            pltpu.VMEM((CHUNK, H_), f32),
            pltpu.VMEM((ZROWS, H_), f32),
            pltpu.VMEM_SHARED((N_PAD, H_), f32),
            pltpu.SemaphoreType.DMA,
            pltpu.SemaphoreType.DMA,
            pltpu.SemaphoreType.DMA,
            pltpu.SemaphoreType.DMA,
        ],
    )

    for _ in range(STEPS_):
        G = gather_call(A, B, dst1, src1)
        (v,) = mlp_call(G, edge_attr, w1e, b1, Wm2, b2, Wm3, b3)
        p0, p1 = scatter_call(v, dst1)
        h, A, B = gru_call(p0, p1, h, W_ih, W_hh, bih, bhh, w1i, w1j)

    head_call = pl.pallas_call(
        _head_body,
        grid=(grid_n,),
        in_specs=[rowblk, full((H_, 1)), full((1, 1))],
        out_specs=[pl.BlockSpec((NB, 1), lambda i: (i, 0))],
        out_shape=[jax.ShapeDtypeStruct((N_, 1), f32)],
    )
    (out,) = head_call(h, W_out, bo)
    return out


# speedup vs baseline: 4.3696x; 1.0783x over previous
import functools

import jax
import jax.numpy as jnp
from jax import lax
from jax.experimental import pallas as pl
from jax.experimental.pallas import tpu as pltpu
from jax.experimental.pallas import tpu_sc as plsc

N_ = 10000
E_ = 320000
H_ = 128
DE_ = 16
STEPS_ = 3

NW = 32
NSPLIT = 2
ESP = E_ // NSPLIT
N_PAD = 10240
TILE_ROWS = N_PAD // 16
ZROWS = 128

NB = 1000
EB = 1280

f32 = jnp.float32


def _init_body(x, w_in, b_in, w1i, w1j, h_o, a_o, b_o):
    h = jnp.dot(x[...], w_in[...], preferred_element_type=f32) + b_in[...]
    h_o[...] = h
    a_o[...] = jnp.dot(h, w1i[...], preferred_element_type=f32)
    b_o[...] = jnp.dot(h, w1j[...], preferred_element_type=f32)


def _mlp_body(g, ea, w1e, b1, w2, b2, w3, b3, v_o):
    t = g[...] + jnp.dot(ea[...], w1e[...], preferred_element_type=f32) + b1[...]
    t = jnp.maximum(t, 0.0)
    u = jnp.maximum(jnp.dot(t, w2[...], preferred_element_type=f32) + b2[...], 0.0)
    v_o[...] = jnp.dot(u, w3[...], preferred_element_type=f32) + b3[...]


def _make_gru_body(nparts):
    def body(*refs):
        ps = refs[:nparts]
        (h, w_ih, w_hh, b_ih, b_hh, w1i, w1j, h_o, a_o, b_o) = refs[nparts:]
        agg = ps[0][...]
        for p in ps[1:]:
            agg = agg + p[...]
        _gru_core(agg, h, w_ih, w_hh, b_ih, b_hh, w1i, w1j, h_o, a_o, b_o)
    return body


def _gru_core(agg, h, w_ih, w_hh, b_ih, b_hh, w1i, w1j, h_o, a_o, b_o):
    hh = h[...]
    gi = jnp.dot(agg, w_ih[...], preferred_element_type=f32) + b_ih[...]
    gh = jnp.dot(hh, w_hh[...], preferred_element_type=f32) + b_hh[...]
    r = jax.nn.sigmoid(gi[:, :H_] + gh[:, :H_])
    z = jax.nn.sigmoid(gi[:, H_:2 * H_] + gh[:, H_:2 * H_])
    n = jnp.tanh(gi[:, 2 * H_:] + r * gh[:, 2 * H_:])
    hn = (1.0 - z) * n + z * hh
    h_o[...] = hn
    a_o[...] = jnp.dot(hn, w1i[...], preferred_element_type=f32)
    b_o[...] = jnp.dot(hn, w1j[...], preferred_element_type=f32)


def _head_body(h, w_out, b_out, o):
    o[...] = jnp.dot(h[...], w_out[...], preferred_element_type=f32) + b_out[...]


def _make_gather_body(e_base, epw, chunk):
    cpw = epw // chunk
    assert cpw % 2 == 1 and chunk % 8 == 0 and cpw * chunk == epw

    def body(a_t, b_t, dst_h, src_h, g_h,
             idx_d, idx_s, abuf0, abuf1, bbuf0, bbuf1, sg0, sg1, so0, so1):
        cid = lax.axis_index("c")
        sid = lax.axis_index("s")
        wid = sid * 2 + cid
        base = e_base + wid * epw
        gbase = wid * epw
        pltpu.sync_copy(dst_h.at[pl.ds(base, epw)], idx_d)
        pltpu.sync_copy(src_h.at[pl.ds(base, epw)], idx_s)

        abufs = (abuf0, abuf1)
        bbufs = (bbuf0, bbuf1)
        sgs = (sg0, sg1)
        sos = (so0, so1)

        def issue(j, p):
            off = j * chunk
            pltpu.async_copy(a_t.at[idx_d.at[pl.ds(off, chunk)]], abufs[p], sgs[p])
            pltpu.async_copy(b_t.at[idx_s.at[pl.ds(off, chunk)]], bbufs[p], sgs[p])

        def wait_gathers(j, p):
            off = j * chunk
            pltpu.make_async_copy(a_t.at[idx_d.at[pl.ds(off, chunk)]], abufs[p], sgs[p]).wait()
            pltpu.make_async_copy(b_t.at[idx_s.at[pl.ds(off, chunk)]], bbufs[p], sgs[p]).wait()

        def drain_out(p):
            pltpu.make_async_copy(abufs[p], g_h.at[pl.ds(gbase, chunk)], sos[p]).wait()

        def add_rows(p):
            ab, bb = abufs[p], bbufs[p]

            def row(i, carry):
                for k in range(8):
                    sl = pl.ds(k * 16, 16)
                    ab[i, sl] = ab[i, sl] + bb[i, sl]
                return carry

            lax.fori_loop(0, chunk, row, 0)

        issue(0, 0)

        def outer(g, carry):
            for b in range(2):
                j = 2 * g + b
                nb = 1 - b

                @pl.when(jnp.logical_or(g >= 1, b == 1))
                def _():
                    drain_out(nb)

                issue(j + 1, nb)
                wait_gathers(j, b)
                add_rows(b)
                pltpu.async_copy(abufs[b], g_h.at[pl.ds(gbase + j * chunk, chunk)], sos[b])
            return carry

        lax.fori_loop(0, (cpw - 1) // 2, outer, 0)

        j_last = cpw - 1
        wait_gathers(j_last, 0)
        add_rows(0)
        pltpu.sync_copy(abufs[0], g_h.at[pl.ds(gbase + j_last * chunk, chunk)])
        drain_out(1)

    return body


def _make_scatter_body(e_base, epw, chunk):
    cpw = epw // chunk
    assert cpw % 2 == 1 and chunk % 8 == 0 and cpw * chunk == epw

    def body(v_h, dst_h, p0_h, p1_h,
             idx0, idx1, vbuf0, vbuf1, zbuf, shared, sv0, sv1, sa0, sa1):
        cid = lax.axis_index("c")
        sid = lax.axis_index("s")
        wid = sid * 2 + cid
        base = e_base + wid * epw
        vbase = wid * epw

        idxs = (idx0, idx1)
        vbufs = (vbuf0, vbuf1)
        svs = (sv0, sv1)
        sas = (sa0, sa1)

        def issue_reads(j, p):
            off = j * chunk
            pltpu.async_copy(v_h.at[pl.ds(vbase + off, chunk)], vbufs[p], svs[p])
            pltpu.async_copy(dst_h.at[pl.ds(base + off, chunk)], idxs[p], svs[p])

        def wait_reads(j, p):
            off = j * chunk
            pltpu.make_async_copy(v_h.at[pl.ds(vbase + off, chunk)], vbufs[p], svs[p]).wait()
            pltpu.make_async_copy(dst_h.at[pl.ds(base + off, chunk)], idxs[p], svs[p]).wait()

        def drain_add(p):
            pltpu.make_async_copy(vbufs[p], shared.at[idxs[p]], sas[p]).wait()

        def zfill(i, carry):
            for k in range(8):
                zbuf[i, pl.ds(k * 16, 16)] = jnp.zeros((16,), f32)
            return carry

        lax.fori_loop(0, ZROWS, zfill, 0)
        issue_reads(0, 0)
        for t in range(TILE_ROWS // ZROWS):
            pltpu.sync_copy(zbuf, shared.at[pl.ds(sid * TILE_ROWS + t * ZROWS, ZROWS)])
        plsc.subcore_barrier()

        def outer(g, carry):
            for b in range(2):
                j = 2 * g + b
                nb = 1 - b

                @pl.when(jnp.logical_or(g >= 1, b == 1))
                def _():
                    drain_add(nb)

                issue_reads(j + 1, nb)
                wait_reads(j, b)
                pltpu.async_copy(vbufs[b], shared.at[idxs[b]], sas[b], add=True)
            return carry

        lax.fori_loop(0, (cpw - 1) // 2, outer, 0)

        j_last = cpw - 1
        drain_add(1)
        wait_reads(j_last, 0)
        pltpu.sync_copy(vbufs[0], shared.at[idxs[0]], add=True)
        plsc.subcore_barrier()

        @pl.when(cid == 0)
        def _():
            for t in range(TILE_ROWS // ZROWS):
                row = sid * TILE_ROWS + t * ZROWS
                pltpu.sync_copy(shared.at[pl.ds(row, ZROWS)], zbuf)
                pltpu.sync_copy(zbuf, p0_h.at[pl.ds(row, ZROWS)])

        @pl.when(cid == 1)
        def _():
            for t in range(TILE_ROWS // ZROWS):
                row = sid * TILE_ROWS + t * ZROWS
                pltpu.sync_copy(shared.at[pl.ds(row, ZROWS)], zbuf)
                pltpu.sync_copy(zbuf, p1_h.at[pl.ds(row, ZROWS)])

    return body


def kernel(x, edge_index, edge_attr, W_in, b_in, Wm1, bm1, Wm2, bm2, Wm3, bm3,
           W_ih, W_hh, b_ih, b_hh, W_out, b_out):
    src1 = edge_index[0]
    dst1 = edge_index[1]
    w1i = Wm1[:H_]
    w1j = Wm1[H_:2 * H_]
    w1e = Wm1[2 * H_:]
    b1 = bm1.reshape(1, H_)
    b2 = bm2.reshape(1, H_ // 2)
    b3 = bm3.reshape(1, H_)
    bih = b_ih.reshape(1, 3 * H_)
    bhh = b_hh.reshape(1, 3 * H_)
    bo = b_out.reshape(1, 1)

    grid_n = N_ // NB

    def full(s):
        return pl.BlockSpec(s, lambda i: (0, 0))

    rowblk = pl.BlockSpec((NB, H_), lambda i: (i, 0))

    init_call = pl.pallas_call(
        _init_body,
        grid=(grid_n,),
        in_specs=[rowblk, full((H_, H_)), full((1, H_)), full((H_, H_)), full((H_, H_))],
        out_specs=[rowblk, rowblk, rowblk],
        out_shape=[jax.ShapeDtypeStruct((N_, H_), f32)] * 3,
    )
    h, A, B = init_call(x, W_in, b_in.reshape(1, H_), w1i, w1j)

    edgeblk = pl.BlockSpec((EB, H_), lambda i: (i, 0))
    mlp_call = pl.pallas_call(
        _mlp_body,
        grid=(ESP // EB,),
        in_specs=[edgeblk, pl.BlockSpec((EB, DE_), lambda i: (i, 0)),
                  full((DE_, H_)), full((1, H_)), full((H_, H_ // 2)),
                  full((1, H_ // 2)), full((H_ // 2, H_)), full((1, H_))],
        out_specs=[edgeblk],
        out_shape=[jax.ShapeDtypeStruct((ESP, H_), f32)],
    )

    gru_call = pl.pallas_call(
        _make_gru_body(2 * NSPLIT),
        grid=(grid_n,),
        in_specs=[rowblk] * (2 * NSPLIT) + [rowblk,
                  full((H_, 3 * H_)), full((H_, 3 * H_)),
                  full((1, 3 * H_)), full((1, 3 * H_)), full((H_, H_)), full((H_, H_))],
        out_specs=[rowblk, rowblk, rowblk],
        out_shape=[jax.ShapeDtypeStruct((N_, H_), f32)] * 3,
    )

    mesh = plsc.VectorSubcoreMesh(core_axis_name="c", subcore_axis_name="s")
    epw_s = ESP // NW
    chunk_s = 40 if NSPLIT == 2 else 80
    gather_calls = []
    scatter_calls = []
    for r in range(NSPLIT):
        gather_calls.append(pl.kernel(
            _make_gather_body(r * ESP, epw_s, chunk_s),
            out_type=jax.ShapeDtypeStruct((ESP, H_), f32),
            mesh=mesh,
            scratch_types=[
                pltpu.VMEM((epw_s,), jnp.int32),
                pltpu.VMEM((epw_s,), jnp.int32),
                pltpu.VMEM((chunk_s, H_), f32),
                pltpu.VMEM((chunk_s, H_), f32),
                pltpu.VMEM((chunk_s, H_), f32),
                pltpu.VMEM((chunk_s, H_), f32),
                pltpu.SemaphoreType.DMA,
                pltpu.SemaphoreType.DMA,
                pltpu.SemaphoreType.DMA,
                pltpu.SemaphoreType.DMA,
            ],
        ))
        scatter_calls.append(pl.kernel(
            _make_scatter_body(r * ESP, epw_s, chunk_s),
            out_type=(jax.ShapeDtypeStruct((N_PAD, H_), f32),
                      jax.ShapeDtypeStruct((N_PAD, H_), f32)),
            mesh=mesh,
            scratch_types=[
                pltpu.VMEM((chunk_s,), jnp.int32),
                pltpu.VMEM((chunk_s,), jnp.int32),
                pltpu.VMEM((chunk_s, H_), f32),
                pltpu.VMEM((chunk_s, H_), f32),
                pltpu.VMEM((ZROWS, H_), f32),
                pltpu.VMEM_SHARED((N_PAD, H_), f32),
                pltpu.SemaphoreType.DMA,
                pltpu.SemaphoreType.DMA,
                pltpu.SemaphoreType.DMA,
                pltpu.SemaphoreType.DMA,
            ],
        ))

    ea_parts = [lax.slice_in_dim(edge_attr, r * ESP, (r + 1) * ESP, axis=0)
                for r in range(NSPLIT)]

    for _ in range(STEPS_):
        gs = [gather_calls[r](A, B, dst1, src1) for r in range(NSPLIT)]
        vs = [mlp_call(gs[r], ea_parts[r], w1e, b1, Wm2, b2, Wm3, b3)[0]
              for r in range(NSPLIT)]
        ps = []
        for r in range(NSPLIT):
            ps.extend(scatter_calls[r](vs[r], dst1))
        h, A, B = gru_call(*ps, h, W_ih, W_hh, bih, bhh, w1i, w1j)

    head_call = pl.pallas_call(
        _head_body,
        grid=(grid_n,),
        in_specs=[rowblk, full((H_, 1)), full((1, 1))],
        out_specs=[pl.BlockSpec((NB, 1), lambda i: (i, 0))],
        out_shape=[jax.ShapeDtypeStruct((N_, 1), f32)],
    )
    (out,) = head_call(h, W_out, bo)
    return out


# speedup vs baseline: 4.5928x; 1.0511x over previous
import functools

import jax
import jax.numpy as jnp
from jax import lax
from jax.experimental import pallas as pl
from jax.experimental.pallas import tpu as pltpu
from jax.experimental.pallas import tpu_sc as plsc

N_ = 10000
E_ = 320000
H_ = 128
DE_ = 16
STEPS_ = 3

NW = 32
NSPLIT = 2
ESP = E_ // NSPLIT
N_PAD = 10240
TILE_ROWS = N_PAD // 16
ZROWS = 128

NB = 1000
EB = 1280

f32 = jnp.float32


def _init_body(x, w_in, b_in, w1i, w1j, h_o, a_o, b_o):
    h = jnp.dot(x[...], w_in[...], preferred_element_type=f32) + b_in[...]
    h_o[...] = h
    a_o[...] = jnp.dot(h, w1i[...], preferred_element_type=f32)
    b_o[...] = jnp.dot(h, w1j[...], preferred_element_type=f32)


def _mlp_body(g, ea, w1e, b1, w2, b2, w3, b3, v_o):
    t = g[...] + jnp.dot(ea[...], w1e[...], preferred_element_type=f32) + b1[...]
    t = jnp.maximum(t, 0.0)
    u = jnp.maximum(jnp.dot(t, w2[...], preferred_element_type=f32) + b2[...], 0.0)
    v_o[...] = jnp.dot(u, w3[...], preferred_element_type=f32) + b3[...]


def _make_gru_body(nparts):
    def body(*refs):
        ps = refs[:nparts]
        (h, w_ih, w_hh, b_ih, b_hh, w1i, w1j, h_o, a_o, b_o) = refs[nparts:]
        agg = ps[0][...]
        for p in ps[1:]:
            agg = agg + p[...]
        _gru_core(agg, h, w_ih, w_hh, b_ih, b_hh, w1i, w1j, h_o, a_o, b_o)
    return body


def _gru_core(agg, h, w_ih, w_hh, b_ih, b_hh, w1i, w1j, h_o, a_o, b_o):
    hh = h[...]
    gi = jnp.dot(agg, w_ih[...], preferred_element_type=f32) + b_ih[...]
    gh = jnp.dot(hh, w_hh[...], preferred_element_type=f32) + b_hh[...]
    r = jax.nn.sigmoid(gi[:, :H_] + gh[:, :H_])
    z = jax.nn.sigmoid(gi[:, H_:2 * H_] + gh[:, H_:2 * H_])
    n = jnp.tanh(gi[:, 2 * H_:] + r * gh[:, 2 * H_:])
    hn = (1.0 - z) * n + z * hh
    h_o[...] = hn
    a_o[...] = jnp.dot(hn, w1i[...], preferred_element_type=f32)
    b_o[...] = jnp.dot(hn, w1j[...], preferred_element_type=f32)


def _head_body(h, w_out, b_out, o):
    o[...] = jnp.dot(h[...], w_out[...], preferred_element_type=f32) + b_out[...]


def _make_gather_call(e_base, epw, chunk, esp, mesh):
    nf = epw // chunk
    tail = epw - nf * chunk
    assert chunk % 8 == 0 and tail % 8 == 0 and nf >= 3

    def body(a_t, b_t, dst_h, src_h, g_h,
             idx_d, idx_s, abuf0, abuf1, bbuf0, bbuf1, sg0, sg1, so0, so1):
        cid = lax.axis_index("c")
        sid = lax.axis_index("s")
        wid = sid * 2 + cid
        base = e_base + wid * epw
        gbase = wid * epw
        pltpu.sync_copy(dst_h.at[pl.ds(base, epw)], idx_d)
        pltpu.sync_copy(src_h.at[pl.ds(base, epw)], idx_s)

        abufs = (abuf0, abuf1)
        bbufs = (bbuf0, bbuf1)
        sgs = (sg0, sg1)
        sos = (so0, so1)

        def issue(j, p, n=chunk, joff=0):
            off = j * chunk + joff
            ab = abufs[p] if n == chunk else abufs[p].at[pl.ds(0, n)]
            bb = bbufs[p] if n == chunk else bbufs[p].at[pl.ds(0, n)]
            pltpu.async_copy(a_t.at[idx_d.at[pl.ds(off, n)]], ab, sgs[p])
            pltpu.async_copy(b_t.at[idx_s.at[pl.ds(off, n)]], bb, sgs[p])

        def wait_gathers(j, p, n=chunk, joff=0):
            off = j * chunk + joff
            ab = abufs[p] if n == chunk else abufs[p].at[pl.ds(0, n)]
            bb = bbufs[p] if n == chunk else bbufs[p].at[pl.ds(0, n)]
            pltpu.make_async_copy(a_t.at[idx_d.at[pl.ds(off, n)]], ab, sgs[p]).wait()
            pltpu.make_async_copy(b_t.at[idx_s.at[pl.ds(off, n)]], bb, sgs[p]).wait()

        def drain_out(p):
            pltpu.make_async_copy(abufs[p], g_h.at[pl.ds(gbase, chunk)], sos[p]).wait()

        def add_rows(p, n=chunk):
            ab, bb = abufs[p], bbufs[p]

            def row(i, carry):
                for k in range(8):
                    sl = pl.ds(k * 16, 16)
                    ab[i, sl] = ab[i, sl] + bb[i, sl]
                return carry

            lax.fori_loop(0, n, row, 0)

        def out_sync(j, p, n=chunk, joff=0):
            src = abufs[p] if n == chunk else abufs[p].at[pl.ds(0, n)]
            pltpu.sync_copy(src, g_h.at[pl.ds(gbase + j * chunk + joff, n)])

        issue(0, 0)
        npairs = (nf - 1) // 2

        def outer(g, carry):
            for b in range(2):
                j = 2 * g + b
                nb = 1 - b

                @pl.when(jnp.logical_or(g >= 1, b == 1))
                def _():
                    drain_out(nb)

                issue(j + 1, nb)
                wait_gathers(j, b)
                add_rows(b)
                pltpu.async_copy(abufs[b], g_h.at[pl.ds(gbase + j * chunk, chunk)], sos[b])
            return carry

        lax.fori_loop(0, npairs, outer, 0)

        if nf % 2 == 1:
            j = nf - 1
            wait_gathers(j, 0)
            add_rows(0)
            if tail:
                drain_out(1)
                issue(nf, 1, n=tail)
                out_sync(j, 0)
                wait_gathers(nf, 1, n=tail)
                add_rows(1, n=tail)
                out_sync(nf, 1, n=tail)
            else:
                out_sync(j, 0)
                drain_out(1)
        else:
            drain_out(1)
            issue(nf - 1, 1)
            wait_gathers(nf - 2, 0)
            add_rows(0)
            pltpu.async_copy(abufs[0], g_h.at[pl.ds(gbase + (nf - 2) * chunk, chunk)], sos[0])
            wait_gathers(nf - 1, 1)
            add_rows(1)
            if tail:
                drain_out(0)
                issue(nf, 0, n=tail)
                out_sync(nf - 1, 1)
                wait_gathers(nf, 0, n=tail)
                add_rows(0, n=tail)
                out_sync(nf, 0, n=tail)
            else:
                out_sync(nf - 1, 1)
                drain_out(0)

    return pl.kernel(
        body,
        out_type=jax.ShapeDtypeStruct((esp, H_), f32),
        mesh=mesh,
        scratch_types=[
            pltpu.VMEM((epw,), jnp.int32),
            pltpu.VMEM((epw,), jnp.int32),
            pltpu.VMEM((chunk, H_), f32),
            pltpu.VMEM((chunk, H_), f32),
            pltpu.VMEM((chunk, H_), f32),
            pltpu.VMEM((chunk, H_), f32),
            pltpu.SemaphoreType.DMA,
            pltpu.SemaphoreType.DMA,
            pltpu.SemaphoreType.DMA,
            pltpu.SemaphoreType.DMA,
        ],
    )


def _make_scatter_call(e_base, epw, chunk, mesh):
    nf = epw // chunk
    tail = epw - nf * chunk
    assert chunk % 8 == 0 and tail % 8 == 0 and nf >= 3

    def body(v_h, dst_h, p0_h, p1_h,
             idx0, idx1, idxt, vbuf0, vbuf1, zbuf, shared, sv0, sv1, sa0, sa1):
        cid = lax.axis_index("c")
        sid = lax.axis_index("s")
        wid = sid * 2 + cid
        base = e_base + wid * epw
        vbase = wid * epw

        idxs = (idx0, idx1)
        vbufs = (vbuf0, vbuf1)
        svs = (sv0, sv1)
        sas = (sa0, sa1)

        def issue_reads(j, p):
            off = j * chunk
            pltpu.async_copy(v_h.at[pl.ds(vbase + off, chunk)], vbufs[p], svs[p])
            pltpu.async_copy(dst_h.at[pl.ds(base + off, chunk)], idxs[p], svs[p])

        def wait_reads(j, p):
            off = j * chunk
            pltpu.make_async_copy(v_h.at[pl.ds(vbase + off, chunk)], vbufs[p], svs[p]).wait()
            pltpu.make_async_copy(dst_h.at[pl.ds(base + off, chunk)], idxs[p], svs[p]).wait()

        def drain_add(p):
            pltpu.make_async_copy(vbufs[p], shared.at[idxs[p]], sas[p]).wait()

        def zfill(i, carry):
            for k in range(8):
                zbuf[i, pl.ds(k * 16, 16)] = jnp.zeros((16,), f32)
            return carry

        lax.fori_loop(0, ZROWS, zfill, 0)
        issue_reads(0, 0)
        for t in range(TILE_ROWS // ZROWS):
            pltpu.sync_copy(zbuf, shared.at[pl.ds(sid * TILE_ROWS + t * ZROWS, ZROWS)])
        plsc.subcore_barrier()

        npairs = (nf - 1) // 2

        def outer(g, carry):
            for b in range(2):
                j = 2 * g + b
                nb = 1 - b

                @pl.when(jnp.logical_or(g >= 1, b == 1))
                def _():
                    drain_add(nb)

                issue_reads(j + 1, nb)
                wait_reads(j, b)
                pltpu.async_copy(vbufs[b], shared.at[idxs[b]], sas[b], add=True)
            return carry

        lax.fori_loop(0, npairs, outer, 0)

        if nf % 2 == 1:
            drain_add(1)
            wait_reads(nf - 1, 0)
            pltpu.sync_copy(vbufs[0], shared.at[idxs[0]], add=True)
        else:
            drain_add(1)
            issue_reads(nf - 1, 1)
            wait_reads(nf - 2, 0)
            pltpu.async_copy(vbufs[0], shared.at[idxs[0]], sas[0], add=True)
            wait_reads(nf - 1, 1)
            pltpu.sync_copy(vbufs[1], shared.at[idxs[1]], add=True)
            drain_add(0)
        if tail:
            off = nf * chunk
            vt = vbufs[0].at[pl.ds(0, tail)]
            pltpu.sync_copy(dst_h.at[pl.ds(base + off, tail)], idxt)
            pltpu.sync_copy(v_h.at[pl.ds(vbase + off, tail)], vt)
            pltpu.sync_copy(vt, shared.at[idxt], add=True)
        plsc.subcore_barrier()

        @pl.when(cid == 0)
        def _():
            for t in range(TILE_ROWS // ZROWS):
                row = sid * TILE_ROWS + t * ZROWS
                pltpu.sync_copy(shared.at[pl.ds(row, ZROWS)], zbuf)
                pltpu.sync_copy(zbuf, p0_h.at[pl.ds(row, ZROWS)])

        @pl.when(cid == 1)
        def _():
            for t in range(TILE_ROWS // ZROWS):
                row = sid * TILE_ROWS + t * ZROWS
                pltpu.sync_copy(shared.at[pl.ds(row, ZROWS)], zbuf)
                pltpu.sync_copy(zbuf, p1_h.at[pl.ds(row, ZROWS)])

    return pl.kernel(
        body,
        out_type=(jax.ShapeDtypeStruct((N_PAD, H_), f32),
                  jax.ShapeDtypeStruct((N_PAD, H_), f32)),
        mesh=mesh,
        scratch_types=[
            pltpu.VMEM((chunk,), jnp.int32),
            pltpu.VMEM((chunk,), jnp.int32),
            pltpu.VMEM((max(tail, 8),), jnp.int32),
            pltpu.VMEM((chunk, H_), f32),
            pltpu.VMEM((chunk, H_), f32),
            pltpu.VMEM((ZROWS, H_), f32),
            pltpu.VMEM_SHARED((N_PAD, H_), f32),
            pltpu.SemaphoreType.DMA,
            pltpu.SemaphoreType.DMA,
            pltpu.SemaphoreType.DMA,
            pltpu.SemaphoreType.DMA,
        ],
    )


def kernel(x, edge_index, edge_attr, W_in, b_in, Wm1, bm1, Wm2, bm2, Wm3, bm3,
           W_ih, W_hh, b_ih, b_hh, W_out, b_out):
    src1 = edge_index[0]
    dst1 = edge_index[1]
    w1i = Wm1[:H_]
    w1j = Wm1[H_:2 * H_]
    w1e = Wm1[2 * H_:]
    b1 = bm1.reshape(1, H_)
    b2 = bm2.reshape(1, H_ // 2)
    b3 = bm3.reshape(1, H_)
    bih = b_ih.reshape(1, 3 * H_)
    bhh = b_hh.reshape(1, 3 * H_)
    bo = b_out.reshape(1, 1)

    grid_n = N_ // NB

    def full(s):
        return pl.BlockSpec(s, lambda i: (0, 0))

    rowblk = pl.BlockSpec((NB, H_), lambda i: (i, 0))

    init_call = pl.pallas_call(
        _init_body,
        grid=(grid_n,),
        in_specs=[rowblk, full((H_, H_)), full((1, H_)), full((H_, H_)), full((H_, H_))],
        out_specs=[rowblk, rowblk, rowblk],
        out_shape=[jax.ShapeDtypeStruct((N_, H_), f32)] * 3,
    )
    h, A, B = init_call(x, W_in, b_in.reshape(1, H_), w1i, w1j)

    edgeblk = pl.BlockSpec((EB, H_), lambda i: (i, 0))
    mlp_call = pl.pallas_call(
        _mlp_body,
        grid=(ESP // EB,),
        in_specs=[edgeblk, pl.BlockSpec((EB, DE_), lambda i: (i, 0)),
                  full((DE_, H_)), full((1, H_)), full((H_, H_ // 2)),
                  full((1, H_ // 2)), full((H_ // 2, H_)), full((1, H_))],
        out_specs=[edgeblk],
        out_shape=[jax.ShapeDtypeStruct((ESP, H_), f32)],
    )

    gru_call = pl.pallas_call(
        _make_gru_body(2 * NSPLIT),
        grid=(grid_n,),
        in_specs=[rowblk] * (2 * NSPLIT) + [rowblk,
                  full((H_, 3 * H_)), full((H_, 3 * H_)),
                  full((1, 3 * H_)), full((1, 3 * H_)), full((H_, H_)), full((H_, H_))],
        out_specs=[rowblk, rowblk, rowblk],
        out_shape=[jax.ShapeDtypeStruct((N_, H_), f32)] * 3,
    )

    mesh = plsc.VectorSubcoreMesh(core_axis_name="c", subcore_axis_name="s")
    epw_s = ESP // NW
    chunk_s = 80
    gather_calls = [_make_gather_call(r * ESP, epw_s, chunk_s, ESP, mesh)
                    for r in range(NSPLIT)]
    scatter_calls = [_make_scatter_call(r * ESP, epw_s, chunk_s, mesh)
                     for r in range(NSPLIT)]

    ea_parts = [lax.slice_in_dim(edge_attr, r * ESP, (r + 1) * ESP, axis=0)
                for r in range(NSPLIT)]

    for _ in range(STEPS_):
        gs = [gather_calls[r](A, B, dst1, src1) for r in range(NSPLIT)]
        vs = [mlp_call(gs[r], ea_parts[r], w1e, b1, Wm2, b2, Wm3, b3)[0]
              for r in range(NSPLIT)]
        ps = []
        for r in range(NSPLIT):
            ps.extend(scatter_calls[r](vs[r], dst1))
        h, A, B = gru_call(*ps, h, W_ih, W_hh, bih, bhh, w1i, w1j)

    head_call = pl.pallas_call(
        _head_body,
        grid=(grid_n,),
        in_specs=[rowblk, full((H_, 1)), full((1, 1))],
        out_specs=[pl.BlockSpec((NB, 1), lambda i: (i, 0))],
        out_shape=[jax.ShapeDtypeStruct((N_, 1), f32)],
    )
    (out,) = head_call(h, W_out, bo)
    return out


# speedup vs baseline: 5.2203x; 1.1366x over previous
import functools

import jax
import jax.numpy as jnp
from jax import lax
from jax.experimental import pallas as pl
from jax.experimental.pallas import tpu as pltpu
from jax.experimental.pallas import tpu_sc as plsc

N_ = 10000
E_ = 320000
H_ = 128
DE_ = 16
STEPS_ = 3

NW = 32
NSPLIT = 2
ESP = E_ // NSPLIT
N_PAD = 10240
TILE_ROWS = N_PAD // 16
ZROWS = 128

NB = 1000
EB = 3200

f32 = jnp.float32


def _init_body(x, w_in, b_in, w1i, w1j, h_o, a_o, b_o):
    h = jnp.dot(x[...], w_in[...], preferred_element_type=f32) + b_in[...]
    h_o[...] = h
    a_o[...] = jnp.dot(h, w1i[...], preferred_element_type=f32)
    b_o[...] = jnp.dot(h, w1j[...], preferred_element_type=f32)


def _mlp_body(g, ea, w1e, b1, w2, b2, w3, b3, v_o):
    t = g[...] + jnp.dot(ea[...], w1e[...], preferred_element_type=f32) + b1[...]
    t = jnp.maximum(t, 0.0)
    u = jnp.maximum(jnp.dot(t, w2[...], preferred_element_type=f32) + b2[...], 0.0)
    v_o[...] = jnp.dot(u, w3[...], preferred_element_type=f32) + b3[...]


def _make_gru_body(nparts):
    def body(*refs):
        ps = refs[:nparts]
        (h, w_ih, w_hh, b_ih, b_hh, w1i, w1j, h_o, a_o, b_o) = refs[nparts:]
        agg = ps[0][...]
        for p in ps[1:]:
            agg = agg + p[...]
        _gru_core(agg, h, w_ih, w_hh, b_ih, b_hh, w1i, w1j, h_o, a_o, b_o)
    return body


def _make_gru_final_body(nparts):
    def body(*refs):
        ps = refs[:nparts]
        (h, w_ih, w_hh, b_ih, b_hh, w_out, b_out, o) = refs[nparts:]
        agg = ps[0][...]
        for p in ps[1:]:
            agg = agg + p[...]
        hn = _gru_hn(agg, h, w_ih, w_hh, b_ih, b_hh)
        o[...] = jnp.dot(hn, w_out[...], preferred_element_type=f32) + b_out[...]
    return body


def _gru_hn(agg, h, w_ih, w_hh, b_ih, b_hh):
    hh = h[...]
    gi = jnp.dot(agg, w_ih[...], preferred_element_type=f32) + b_ih[...]
    gh = jnp.dot(hh, w_hh[...], preferred_element_type=f32) + b_hh[...]
    r = jax.nn.sigmoid(gi[:, :H_] + gh[:, :H_])
    z = jax.nn.sigmoid(gi[:, H_:2 * H_] + gh[:, H_:2 * H_])
    n = jnp.tanh(gi[:, 2 * H_:] + r * gh[:, 2 * H_:])
    return (1.0 - z) * n + z * hh


def _gru_core(agg, h, w_ih, w_hh, b_ih, b_hh, w1i, w1j, h_o, a_o, b_o):
    hn = _gru_hn(agg, h, w_ih, w_hh, b_ih, b_hh)
    h_o[...] = hn
    a_o[...] = jnp.dot(hn, w1i[...], preferred_element_type=f32)
    b_o[...] = jnp.dot(hn, w1j[...], preferred_element_type=f32)


def _head_body(h, w_out, b_out, o):
    o[...] = jnp.dot(h[...], w_out[...], preferred_element_type=f32) + b_out[...]


def _make_gather_call(e_base, epw, chunk, esp, mesh):
    nf = epw // chunk
    tail = epw - nf * chunk
    assert chunk % 8 == 0 and tail % 8 == 0 and nf >= 3

    def body(a_t, b_t, dst_h, src_h, g_h,
             idx_d, idx_s, abuf0, abuf1, bbuf0, bbuf1, sg0, sg1, so0, so1):
        cid = lax.axis_index("c")
        sid = lax.axis_index("s")
        wid = sid * 2 + cid
        base = e_base + wid * epw
        gbase = wid * epw
        pltpu.sync_copy(dst_h.at[pl.ds(base, epw)], idx_d)
        pltpu.sync_copy(src_h.at[pl.ds(base, epw)], idx_s)

        abufs = (abuf0, abuf1)
        bbufs = (bbuf0, bbuf1)
        sgs = (sg0, sg1)
        sos = (so0, so1)

        def issue(j, p, n=chunk, joff=0):
            off = j * chunk + joff
            ab = abufs[p] if n == chunk else abufs[p].at[pl.ds(0, n)]
            bb = bbufs[p] if n == chunk else bbufs[p].at[pl.ds(0, n)]
            pltpu.async_copy(a_t.at[idx_d.at[pl.ds(off, n)]], ab, sgs[p])
            pltpu.async_copy(b_t.at[idx_s.at[pl.ds(off, n)]], bb, sgs[p])

        def wait_gathers(j, p, n=chunk, joff=0):
            off = j * chunk + joff
            ab = abufs[p] if n == chunk else abufs[p].at[pl.ds(0, n)]
            bb = bbufs[p] if n == chunk else bbufs[p].at[pl.ds(0, n)]
            pltpu.make_async_copy(a_t.at[idx_d.at[pl.ds(off, n)]], ab, sgs[p]).wait()
            pltpu.make_async_copy(b_t.at[idx_s.at[pl.ds(off, n)]], bb, sgs[p]).wait()

        def drain_out(p):
            pltpu.make_async_copy(abufs[p], g_h.at[pl.ds(gbase, chunk)], sos[p]).wait()

        def add_rows(p, n=chunk):
            ab, bb = abufs[p], bbufs[p]

            def row(i, carry):
                for k in range(8):
                    sl = pl.ds(k * 16, 16)
                    ab[i, sl] = ab[i, sl] + bb[i, sl]
                return carry

            lax.fori_loop(0, n, row, 0)

        def out_sync(j, p, n=chunk, joff=0):
            src = abufs[p] if n == chunk else abufs[p].at[pl.ds(0, n)]
            pltpu.sync_copy(src, g_h.at[pl.ds(gbase + j * chunk + joff, n)])

        issue(0, 0)
        npairs = (nf - 1) // 2

        def outer(g, carry):
            for b in range(2):
                j = 2 * g + b
                nb = 1 - b

                @pl.when(jnp.logical_or(g >= 1, b == 1))
                def _():
                    drain_out(nb)

                issue(j + 1, nb)
                wait_gathers(j, b)
                add_rows(b)
                pltpu.async_copy(abufs[b], g_h.at[pl.ds(gbase + j * chunk, chunk)], sos[b])
            return carry

        lax.fori_loop(0, npairs, outer, 0)

        if nf % 2 == 1:
            j = nf - 1
            wait_gathers(j, 0)
            add_rows(0)
            if tail:
                drain_out(1)
                issue(nf, 1, n=tail)
                out_sync(j, 0)
                wait_gathers(nf, 1, n=tail)
                add_rows(1, n=tail)
                out_sync(nf, 1, n=tail)
            else:
                out_sync(j, 0)
                drain_out(1)
        else:
            drain_out(1)
            issue(nf - 1, 1)
            wait_gathers(nf - 2, 0)
            add_rows(0)
            pltpu.async_copy(abufs[0], g_h.at[pl.ds(gbase + (nf - 2) * chunk, chunk)], sos[0])
            wait_gathers(nf - 1, 1)
            add_rows(1)
            if tail:
                drain_out(0)
                issue(nf, 0, n=tail)
                out_sync(nf - 1, 1)
                wait_gathers(nf, 0, n=tail)
                add_rows(0, n=tail)
                out_sync(nf, 0, n=tail)
            else:
                out_sync(nf - 1, 1)
                drain_out(0)

    return pl.kernel(
        body,
        out_type=jax.ShapeDtypeStruct((esp, H_), f32),
        mesh=mesh,
        scratch_types=[
            pltpu.VMEM((epw,), jnp.int32),
            pltpu.VMEM((epw,), jnp.int32),
            pltpu.VMEM((chunk, H_), f32),
            pltpu.VMEM((chunk, H_), f32),
            pltpu.VMEM((chunk, H_), f32),
            pltpu.VMEM((chunk, H_), f32),
            pltpu.SemaphoreType.DMA,
            pltpu.SemaphoreType.DMA,
            pltpu.SemaphoreType.DMA,
            pltpu.SemaphoreType.DMA,
        ],
    )


def _make_scatter_call(e_base, epw, chunk, mesh):
    nf = epw // chunk
    tail = epw - nf * chunk
    assert chunk % 8 == 0 and tail % 8 == 0 and nf >= 3

    def body(v_h, dst_h, p0_h, p1_h,
             idx0, idx1, idxt, vbuf0, vbuf1, zbuf, shared, sv0, sv1, sa0, sa1):
        cid = lax.axis_index("c")
        sid = lax.axis_index("s")
        wid = sid * 2 + cid
        base = e_base + wid * epw
        vbase = wid * epw

        idxs = (idx0, idx1)
        vbufs = (vbuf0, vbuf1)
        svs = (sv0, sv1)
        sas = (sa0, sa1)

        def issue_reads(j, p):
            off = j * chunk
            pltpu.async_copy(v_h.at[pl.ds(vbase + off, chunk)], vbufs[p], svs[p])
            pltpu.async_copy(dst_h.at[pl.ds(base + off, chunk)], idxs[p], svs[p])

        def wait_reads(j, p):
            off = j * chunk
            pltpu.make_async_copy(v_h.at[pl.ds(vbase + off, chunk)], vbufs[p], svs[p]).wait()
            pltpu.make_async_copy(dst_h.at[pl.ds(base + off, chunk)], idxs[p], svs[p]).wait()

        def drain_add(p):
            pltpu.make_async_copy(vbufs[p], shared.at[idxs[p]], sas[p]).wait()

        def zfill(i, carry):
            for k in range(8):
                zbuf[i, pl.ds(k * 16, 16)] = jnp.zeros((16,), f32)
            return carry

        lax.fori_loop(0, ZROWS, zfill, 0)
        issue_reads(0, 0)
        for t in range(TILE_ROWS // ZROWS):
            pltpu.sync_copy(zbuf, shared.at[pl.ds(sid * TILE_ROWS + t * ZROWS, ZROWS)])
        plsc.subcore_barrier()

        npairs = (nf - 1) // 2

        def outer(g, carry):
            for b in range(2):
                j = 2 * g + b
                nb = 1 - b

                @pl.when(jnp.logical_or(g >= 1, b == 1))
                def _():
                    drain_add(nb)

                issue_reads(j + 1, nb)
                wait_reads(j, b)
                pltpu.async_copy(vbufs[b], shared.at[idxs[b]], sas[b], add=True)
            return carry

        lax.fori_loop(0, npairs, outer, 0)

        if nf % 2 == 1:
            drain_add(1)
            wait_reads(nf - 1, 0)
            pltpu.sync_copy(vbufs[0], shared.at[idxs[0]], add=True)
        else:
            drain_add(1)
            issue_reads(nf - 1, 1)
            wait_reads(nf - 2, 0)
            pltpu.async_copy(vbufs[0], shared.at[idxs[0]], sas[0], add=True)
            wait_reads(nf - 1, 1)
            pltpu.sync_copy(vbufs[1], shared.at[idxs[1]], add=True)
            drain_add(0)
        if tail:
            off = nf * chunk
            vt = vbufs[0].at[pl.ds(0, tail)]
            pltpu.sync_copy(dst_h.at[pl.ds(base + off, tail)], idxt)
            pltpu.sync_copy(v_h.at[pl.ds(vbase + off, tail)], vt)
            pltpu.sync_copy(vt, shared.at[idxt], add=True)
        plsc.subcore_barrier()

        @pl.when(cid == 0)
        def _():
            for t in range(TILE_ROWS // ZROWS):
                row = sid * TILE_ROWS + t * ZROWS
                pltpu.sync_copy(shared.at[pl.ds(row, ZROWS)], p0_h.at[pl.ds(row, ZROWS)])

        @pl.when(cid == 1)
        def _():
            for t in range(TILE_ROWS // ZROWS):
                row = sid * TILE_ROWS + t * ZROWS
                pltpu.sync_copy(shared.at[pl.ds(row, ZROWS)], p1_h.at[pl.ds(row, ZROWS)])

    return pl.kernel(
        body,
        out_type=(jax.ShapeDtypeStruct((N_PAD, H_), f32),
                  jax.ShapeDtypeStruct((N_PAD, H_), f32)),
        mesh=mesh,
        scratch_types=[
            pltpu.VMEM((chunk,), jnp.int32),
            pltpu.VMEM((chunk,), jnp.int32),
            pltpu.VMEM((max(tail, 8),), jnp.int32),
            pltpu.VMEM((chunk, H_), f32),
            pltpu.VMEM((chunk, H_), f32),
            pltpu.VMEM((ZROWS, H_), f32),
            pltpu.VMEM_SHARED((N_PAD, H_), f32),
            pltpu.SemaphoreType.DMA,
            pltpu.SemaphoreType.DMA,
            pltpu.SemaphoreType.DMA,
            pltpu.SemaphoreType.DMA,
        ],
    )


def kernel(x, edge_index, edge_attr, W_in, b_in, Wm1, bm1, Wm2, bm2, Wm3, bm3,
           W_ih, W_hh, b_ih, b_hh, W_out, b_out):
    src1 = edge_index[0]
    dst1 = edge_index[1]
    w1i = Wm1[:H_]
    w1j = Wm1[H_:2 * H_]
    w1e = Wm1[2 * H_:]
    b1 = bm1.reshape(1, H_)
    b2 = bm2.reshape(1, H_ // 2)
    b3 = bm3.reshape(1, H_)
    bih = b_ih.reshape(1, 3 * H_)
    bhh = b_hh.reshape(1, 3 * H_)
    bo = b_out.reshape(1, 1)

    grid_n = N_ // NB

    def full(s):
        return pl.BlockSpec(s, lambda i: (0, 0))

    rowblk = pl.BlockSpec((NB, H_), lambda i: (i, 0))

    init_call = pl.pallas_call(
        _init_body,
        grid=(grid_n,),
        in_specs=[rowblk, full((H_, H_)), full((1, H_)), full((H_, H_)), full((H_, H_))],
        out_specs=[rowblk, rowblk, rowblk],
        out_shape=[jax.ShapeDtypeStruct((N_, H_), f32)] * 3,
    )
    h, A, B = init_call(x, W_in, b_in.reshape(1, H_), w1i, w1j)

    edgeblk = pl.BlockSpec((EB, H_), lambda i: (i, 0))
    mlp_call = pl.pallas_call(
        _mlp_body,
        grid=(ESP // EB,),
        in_specs=[edgeblk, pl.BlockSpec((EB, DE_), lambda i: (i, 0)),
                  full((DE_, H_)), full((1, H_)), full((H_, H_ // 2)),
                  full((1, H_ // 2)), full((H_ // 2, H_)), full((1, H_))],
        out_specs=[edgeblk],
        out_shape=[jax.ShapeDtypeStruct((ESP, H_), f32)],
    )

    gru_call = pl.pallas_call(
        _make_gru_body(2 * NSPLIT),
        grid=(grid_n,),
        in_specs=[rowblk] * (2 * NSPLIT) + [rowblk,
                  full((H_, 3 * H_)), full((H_, 3 * H_)),
                  full((1, 3 * H_)), full((1, 3 * H_)), full((H_, H_)), full((H_, H_))],
        out_specs=[rowblk, rowblk, rowblk],
        out_shape=[jax.ShapeDtypeStruct((N_, H_), f32)] * 3,
    )

    mesh = plsc.VectorSubcoreMesh(core_axis_name="c", subcore_axis_name="s")
    epw_s = ESP // NW
    chunk_s = 80
    gather_calls = [_make_gather_call(r * ESP, epw_s, chunk_s, ESP, mesh)
                    for r in range(NSPLIT)]
    scatter_calls = [_make_scatter_call(r * ESP, epw_s, chunk_s, mesh)
                     for r in range(NSPLIT)]

    gru_final_call = pl.pallas_call(
        _make_gru_final_body(2 * NSPLIT),
        grid=(grid_n,),
        in_specs=[rowblk] * (2 * NSPLIT) + [rowblk,
                  full((H_, 3 * H_)), full((H_, 3 * H_)),
                  full((1, 3 * H_)), full((1, 3 * H_)), full((H_, 1)), full((1, 1))],
        out_specs=[pl.BlockSpec((NB, 1), lambda i: (i, 0))],
        out_shape=[jax.ShapeDtypeStruct((N_, 1), f32)],
    )

    ea_parts = [lax.slice_in_dim(edge_attr, r * ESP, (r + 1) * ESP, axis=0)
                for r in range(NSPLIT)]

    for step in range(STEPS_):
        gs = [gather_calls[r](A, B, dst1, src1) for r in range(NSPLIT)]
        vs = [mlp_call(gs[r], ea_parts[r], w1e, b1, Wm2, b2, Wm3, b3)[0]
              for r in range(NSPLIT)]
        ps = []
        for r in range(NSPLIT):
            ps.extend(scatter_calls[r](vs[r], dst1))
        if step < STEPS_ - 1:
            h, A, B = gru_call(*ps, h, W_ih, W_hh, bih, bhh, w1i, w1j)
        else:
            (out,) = gru_final_call(*ps, h, W_ih, W_hh, bih, bhh, W_out, bo)
    return out


# speedup vs baseline: 5.2821x; 1.0119x over previous
import functools

import jax
import jax.numpy as jnp
from jax import lax
from jax.experimental import pallas as pl
from jax.experimental.pallas import tpu as pltpu
from jax.experimental.pallas import tpu_sc as plsc

N_ = 10000
E_ = 320000
H_ = 128
DE_ = 16
STEPS_ = 3

NW = 32
NSPLIT = 2
ESP = E_ // NSPLIT
N_PAD = 10240
TILE_ROWS = N_PAD // 16
ZROWS = 128

NB = 1000
EB = 3200

f32 = jnp.float32


def _init_body(x, w_in, b_in, w1i, w1j, h_o, a_o, b_o):
    h = jnp.dot(x[...], w_in[...], preferred_element_type=f32) + b_in[...]
    h_o[...] = h
    a_o[...] = jnp.dot(h, w1i[...], preferred_element_type=f32)
    b_o[...] = jnp.dot(h, w1j[...], preferred_element_type=f32)


def _mlp_body(g, ea, w1e, b1, w2, b2, w3, b3, v_o):
    t = g[...] + jnp.dot(ea[...], w1e[...], preferred_element_type=f32) + b1[...]
    t = jnp.maximum(t, 0.0)
    u = jnp.maximum(jnp.dot(t, w2[...], preferred_element_type=f32) + b2[...], 0.0)
    v_o[...] = jnp.dot(u, w3[...], preferred_element_type=f32) + b3[...]


def _make_gru_body(nparts):
    def body(*refs):
        ps = refs[:nparts]
        (h, w_ih, w_hh, b_ih, b_hh, w1i, w1j, h_o, a_o, b_o) = refs[nparts:]
        agg = ps[0][...]
        for p in ps[1:]:
            agg = agg + p[...]
        _gru_core(agg, h, w_ih, w_hh, b_ih, b_hh, w1i, w1j, h_o, a_o, b_o)
    return body


def _make_gru_final_body(nparts):
    def body(*refs):
        ps = refs[:nparts]
        (h, w_ih, w_hh, b_ih, b_hh, w_out, b_out, o) = refs[nparts:]
        agg = ps[0][...]
        for p in ps[1:]:
            agg = agg + p[...]
        hn = _gru_hn(agg, h, w_ih, w_hh, b_ih, b_hh)
        o[...] = jnp.dot(hn, w_out[...], preferred_element_type=f32) + b_out[...]
    return body


def _gru_hn(agg, h, w_ih, w_hh, b_ih, b_hh):
    hh = h[...]
    gi = jnp.dot(agg, w_ih[...], preferred_element_type=f32) + b_ih[...]
    gh = jnp.dot(hh, w_hh[...], preferred_element_type=f32) + b_hh[...]
    r = jax.nn.sigmoid(gi[:, :H_] + gh[:, :H_])
    z = jax.nn.sigmoid(gi[:, H_:2 * H_] + gh[:, H_:2 * H_])
    n = jnp.tanh(gi[:, 2 * H_:] + r * gh[:, 2 * H_:])
    return (1.0 - z) * n + z * hh


def _gru_core(agg, h, w_ih, w_hh, b_ih, b_hh, w1i, w1j, h_o, a_o, b_o):
    hn = _gru_hn(agg, h, w_ih, w_hh, b_ih, b_hh)
    h_o[...] = hn
    a_o[...] = jnp.dot(hn, w1i[...], preferred_element_type=f32)
    b_o[...] = jnp.dot(hn, w1j[...], preferred_element_type=f32)


def _head_body(h, w_out, b_out, o):
    o[...] = jnp.dot(h[...], w_out[...], preferred_element_type=f32) + b_out[...]


def _make_gather_call(e_base, epw, chunk, esp, mesh):
    nf = epw // chunk
    tail = epw - nf * chunk
    assert chunk % 8 == 0 and tail % 8 == 0 and nf >= 3

    def body(a_t, b_t, dst_h, src_h, g_h,
             idx_d, idx_s, abuf0, abuf1, bbuf0, bbuf1, obuf0, obuf1,
             sg0, sg1, so0, so1):
        cid = lax.axis_index("c")
        sid = lax.axis_index("s")
        wid = sid * 2 + cid
        base = e_base + wid * epw
        gbase = wid * epw
        pltpu.sync_copy(dst_h.at[pl.ds(base, epw)], idx_d)
        pltpu.sync_copy(src_h.at[pl.ds(base, epw)], idx_s)

        abufs = (abuf0, abuf1)
        bbufs = (bbuf0, bbuf1)
        obufs = (obuf0, obuf1)
        sgs = (sg0, sg1)
        sos = (so0, so1)

        def issue(j, p, n=chunk):
            off = j * chunk
            ab = abufs[p] if n == chunk else abufs[p].at[pl.ds(0, n)]
            bb = bbufs[p] if n == chunk else bbufs[p].at[pl.ds(0, n)]
            pltpu.async_copy(a_t.at[idx_d.at[pl.ds(off, n)]], ab, sgs[p])
            pltpu.async_copy(b_t.at[idx_s.at[pl.ds(off, n)]], bb, sgs[p])

        def wait_gathers(j, p, n=chunk):
            off = j * chunk
            ab = abufs[p] if n == chunk else abufs[p].at[pl.ds(0, n)]
            bb = bbufs[p] if n == chunk else bbufs[p].at[pl.ds(0, n)]
            pltpu.make_async_copy(a_t.at[idx_d.at[pl.ds(off, n)]], ab, sgs[p]).wait()
            pltpu.make_async_copy(b_t.at[idx_s.at[pl.ds(off, n)]], bb, sgs[p]).wait()

        def drain_out(p):
            pltpu.make_async_copy(obufs[p], g_h.at[pl.ds(gbase, chunk)], sos[p]).wait()

        def add_rows(p, n=chunk):
            ab, bb, ob = abufs[p], bbufs[p], obufs[p]

            def row(i, carry):
                for k in range(8):
                    sl = pl.ds(k * 16, 16)
                    ob[i, sl] = ab[i, sl] + bb[i, sl]
                return carry

            lax.fori_loop(0, n, row, 0)

        def out_sync(j, p, n=chunk):
            src = obufs[p] if n == chunk else obufs[p].at[pl.ds(0, n)]
            pltpu.sync_copy(src, g_h.at[pl.ds(gbase + j * chunk, n)])

        def out_async(j, p):
            pltpu.async_copy(obufs[p], g_h.at[pl.ds(gbase + j * chunk, chunk)], sos[p])

        issue(0, 0)
        npairs = (nf - 1) // 2

        def outer(g, carry):
            for b in range(2):
                j = 2 * g + b

                issue(j + 1, 1 - b)
                wait_gathers(j, b)

                @pl.when(g >= 1)
                def _():
                    drain_out(b)

                add_rows(b)
                out_async(j, b)
            return carry

        lax.fori_loop(0, npairs, outer, 0)

        nrem = nf - 2 * npairs
        jr = 2 * npairs
        if nrem == 2:
            issue(jr + 1, 1)
        for k in range(nrem):
            j = jr + k
            p = j % 2
            wait_gathers(j, p)
            drain_out(p)
            add_rows(p)
            out_async(j, p)
        if tail:
            pt = nf % 2
            issue(nf, pt, n=tail)
            wait_gathers(nf, pt, n=tail)
            drain_out(pt)
            add_rows(pt, n=tail)
            out_sync(nf, pt, n=tail)
            drain_out(1 - pt)
        else:
            drain_out(0)
            drain_out(1)

    return pl.kernel(
        body,
        out_type=jax.ShapeDtypeStruct((esp, H_), f32),
        mesh=mesh,
        scratch_types=[
            pltpu.VMEM((epw,), jnp.int32),
            pltpu.VMEM((epw,), jnp.int32),
            pltpu.VMEM((chunk, H_), f32),
            pltpu.VMEM((chunk, H_), f32),
            pltpu.VMEM((chunk, H_), f32),
            pltpu.VMEM((chunk, H_), f32),
            pltpu.VMEM((chunk, H_), f32),
            pltpu.VMEM((chunk, H_), f32),
            pltpu.SemaphoreType.DMA,
            pltpu.SemaphoreType.DMA,
            pltpu.SemaphoreType.DMA,
            pltpu.SemaphoreType.DMA,
        ],
    )


def _make_scatter_call(e_base, epw, chunk, mesh):
    nf = epw // chunk
    tail = epw - nf * chunk
    assert chunk % 8 == 0 and tail % 8 == 0 and nf >= 3

    def body(v_h, dst_h, p0_h, p1_h,
             idx0, idx1, idxt, vbuf0, vbuf1, zbuf, shared, sv0, sv1, sa0, sa1):
        cid = lax.axis_index("c")
        sid = lax.axis_index("s")
        wid = sid * 2 + cid
        base = e_base + wid * epw
        vbase = wid * epw

        idxs = (idx0, idx1)
        vbufs = (vbuf0, vbuf1)
        svs = (sv0, sv1)
        sas = (sa0, sa1)

        def issue_reads(j, p):
            off = j * chunk
            pltpu.async_copy(v_h.at[pl.ds(vbase + off, chunk)], vbufs[p], svs[p])
            pltpu.async_copy(dst_h.at[pl.ds(base + off, chunk)], idxs[p], svs[p])

        def wait_reads(j, p):
            off = j * chunk
            pltpu.make_async_copy(v_h.at[pl.ds(vbase + off, chunk)], vbufs[p], svs[p]).wait()
            pltpu.make_async_copy(dst_h.at[pl.ds(base + off, chunk)], idxs[p], svs[p]).wait()

        def drain_add(p):
            pltpu.make_async_copy(vbufs[p], shared.at[idxs[p]], sas[p]).wait()

        def zfill(i, carry):
            for k in range(8):
                zbuf[i, pl.ds(k * 16, 16)] = jnp.zeros((16,), f32)
            return carry

        lax.fori_loop(0, ZROWS, zfill, 0)
        issue_reads(0, 0)
        for t in range(TILE_ROWS // ZROWS):
            pltpu.sync_copy(zbuf, shared.at[pl.ds(sid * TILE_ROWS + t * ZROWS, ZROWS)])
        plsc.subcore_barrier()

        npairs = (nf - 1) // 2

        def outer(g, carry):
            for b in range(2):
                j = 2 * g + b
                nb = 1 - b

                @pl.when(jnp.logical_or(g >= 1, b == 1))
                def _():
                    drain_add(nb)

                issue_reads(j + 1, nb)
                wait_reads(j, b)
                pltpu.async_copy(vbufs[b], shared.at[idxs[b]], sas[b], add=True)
            return carry

        lax.fori_loop(0, npairs, outer, 0)

        if nf % 2 == 1:
            drain_add(1)
            wait_reads(nf - 1, 0)
            pltpu.sync_copy(vbufs[0], shared.at[idxs[0]], add=True)
        else:
            drain_add(1)
            issue_reads(nf - 1, 1)
            wait_reads(nf - 2, 0)
            pltpu.async_copy(vbufs[0], shared.at[idxs[0]], sas[0], add=True)
            wait_reads(nf - 1, 1)
            pltpu.sync_copy(vbufs[1], shared.at[idxs[1]], add=True)
            drain_add(0)
        if tail:
            off = nf * chunk
            vt = vbufs[0].at[pl.ds(0, tail)]
            pltpu.sync_copy(dst_h.at[pl.ds(base + off, tail)], idxt)
            pltpu.sync_copy(v_h.at[pl.ds(vbase + off, tail)], vt)
            pltpu.sync_copy(vt, shared.at[idxt], add=True)
        plsc.subcore_barrier()

        @pl.when(cid == 0)
        def _():
            for t in range(TILE_ROWS // ZROWS):
                row = sid * TILE_ROWS + t * ZROWS
                pltpu.sync_copy(shared.at[pl.ds(row, ZROWS)], p0_h.at[pl.ds(row, ZROWS)])

        @pl.when(cid == 1)
        def _():
            for t in range(TILE_ROWS // ZROWS):
                row = sid * TILE_ROWS + t * ZROWS
                pltpu.sync_copy(shared.at[pl.ds(row, ZROWS)], p1_h.at[pl.ds(row, ZROWS)])

    return pl.kernel(
        body,
        out_type=(jax.ShapeDtypeStruct((N_PAD, H_), f32),
                  jax.ShapeDtypeStruct((N_PAD, H_), f32)),
        mesh=mesh,
        scratch_types=[
            pltpu.VMEM((chunk,), jnp.int32),
            pltpu.VMEM((chunk,), jnp.int32),
            pltpu.VMEM((max(tail, 8),), jnp.int32),
            pltpu.VMEM((chunk, H_), f32),
            pltpu.VMEM((chunk, H_), f32),
            pltpu.VMEM((ZROWS, H_), f32),
            pltpu.VMEM_SHARED((N_PAD, H_), f32),
            pltpu.SemaphoreType.DMA,
            pltpu.SemaphoreType.DMA,
            pltpu.SemaphoreType.DMA,
            pltpu.SemaphoreType.DMA,
        ],
    )


def kernel(x, edge_index, edge_attr, W_in, b_in, Wm1, bm1, Wm2, bm2, Wm3, bm3,
           W_ih, W_hh, b_ih, b_hh, W_out, b_out):
    src1 = edge_index[0]
    dst1 = edge_index[1]
    w1i = Wm1[:H_]
    w1j = Wm1[H_:2 * H_]
    w1e = Wm1[2 * H_:]
    b1 = bm1.reshape(1, H_)
    b2 = bm2.reshape(1, H_ // 2)
    b3 = bm3.reshape(1, H_)
    bih = b_ih.reshape(1, 3 * H_)
    bhh = b_hh.reshape(1, 3 * H_)
    bo = b_out.reshape(1, 1)

    grid_n = N_ // NB

    def full(s):
        return pl.BlockSpec(s, lambda i: (0, 0))

    rowblk = pl.BlockSpec((NB, H_), lambda i: (i, 0))

    init_call = pl.pallas_call(
        _init_body,
        grid=(grid_n,),
        in_specs=[rowblk, full((H_, H_)), full((1, H_)), full((H_, H_)), full((H_, H_))],
        out_specs=[rowblk, rowblk, rowblk],
        out_shape=[jax.ShapeDtypeStruct((N_, H_), f32)] * 3,
    )
    h, A, B = init_call(x, W_in, b_in.reshape(1, H_), w1i, w1j)

    edgeblk = pl.BlockSpec((EB, H_), lambda i: (i, 0))
    mlp_call = pl.pallas_call(
        _mlp_body,
        grid=(ESP // EB,),
        in_specs=[edgeblk, pl.BlockSpec((EB, DE_), lambda i: (i, 0)),
                  full((DE_, H_)), full((1, H_)), full((H_, H_ // 2)),
                  full((1, H_ // 2)), full((H_ // 2, H_)), full((1, H_))],
        out_specs=[edgeblk],
        out_shape=[jax.ShapeDtypeStruct((ESP, H_), f32)],
    )

    gru_call = pl.pallas_call(
        _make_gru_body(2 * NSPLIT),
        grid=(grid_n,),
        in_specs=[rowblk] * (2 * NSPLIT) + [rowblk,
                  full((H_, 3 * H_)), full((H_, 3 * H_)),
                  full((1, 3 * H_)), full((1, 3 * H_)), full((H_, H_)), full((H_, H_))],
        out_specs=[rowblk, rowblk, rowblk],
        out_shape=[jax.ShapeDtypeStruct((N_, H_), f32)] * 3,
    )

    mesh = plsc.VectorSubcoreMesh(core_axis_name="c", subcore_axis_name="s")
    epw_s = ESP // NW
    gather_calls = [_make_gather_call(r * ESP, epw_s, 128, ESP, mesh)
                    for r in range(NSPLIT)]
    scatter_calls = [_make_scatter_call(r * ESP, epw_s, 80, mesh)
                     for r in range(NSPLIT)]

    gru_final_call = pl.pallas_call(
        _make_gru_final_body(2 * NSPLIT),
        grid=(grid_n,),
        in_specs=[rowblk] * (2 * NSPLIT) + [rowblk,
                  full((H_, 3 * H_)), full((H_, 3 * H_)),
                  full((1, 3 * H_)), full((1, 3 * H_)), full((H_, 1)), full((1, 1))],
        out_specs=[pl.BlockSpec((NB, 1), lambda i: (i, 0))],
        out_shape=[jax.ShapeDtypeStruct((N_, 1), f32)],
    )

    ea_parts = [lax.slice_in_dim(edge_attr, r * ESP, (r + 1) * ESP, axis=0)
                for r in range(NSPLIT)]

    for step in range(STEPS_):
        gs = [gather_calls[r](A, B, dst1, src1) for r in range(NSPLIT)]
        vs = [mlp_call(gs[r], ea_parts[r], w1e, b1, Wm2, b2, Wm3, b3)[0]
              for r in range(NSPLIT)]
        ps = []
        for r in range(NSPLIT):
            ps.extend(scatter_calls[r](vs[r], dst1))
        if step < STEPS_ - 1:
            h, A, B = gru_call(*ps, h, W_ih, W_hh, bih, bhh, w1i, w1j)
        else:
            (out,) = gru_final_call(*ps, h, W_ih, W_hh, bih, bhh, W_out, bo)
    return out
